# baseline (speedup 1.0000x reference)
import jax
import jax.numpy as jnp
from jax import lax
from jax.experimental import pallas as pl
from jax.experimental.pallas import tpu as pltpu
from jax.experimental.pallas import tpu_sc as plsc

B = 16384
D = 64
V = 1_000_000
VB = 128
NB_TOT = (V + VB - 1) // VB
NC = 2
NS = 16
NW = NC * NS
NBW = (NB_TOT + NW - 1) // NW
CAP = 16
L = 16
SS = 4
SW = SS * VB
RING = 3


def _lane(v, i):
    sel = jnp.where(lax.iota(jnp.int32, L) == i, v, 0)
    return lax.reduce_max(sel, (0,))


def _body(ids_hbm, table_hbm, out_hbm, aid_v, bk_ids, bk_pos, cnt_v,
          blk, rb, gsem, osem0, osem1):
    wid = lax.axis_index("s") * NC + lax.axis_index("c")
    lo = wid * NBW
    hi = jnp.minimum(lo + NBW, NB_TOT)
    nb = hi - lo
    iota = lax.iota(jnp.int32, L)

    nss = (nb + SS - 1) // SS

    def ssbase(g):
        return jnp.where(g == nss - 1, nb - SS, g * SS)

    def fire(g):
        r = lax.rem(g, RING)
        off = pl.multiple_of((lo + ssbase(g)) * VB, VB)
        for i in range(D // 8):
            pltpu.async_copy(
                table_hbm.at[pl.ds(i * 8, 8), pl.ds(off, SW)],
                blk.at[r, pl.ds(i * 8, 8), :], gsem.at[r])
        return None

    def slab_wait(g):
        r = lax.rem(g, RING)
        for i in range(D // 8):
            pltpu.make_async_copy(
                table_hbm.at[pl.ds(0, 8), pl.ds(0, SW)],
                blk.at[r, pl.ds(0, 8), :], gsem.at[r]).wait()
        return None

    for gg in range(RING - 1):
        fire(gg)

    pltpu.sync_copy(ids_hbm, aid_v)

    def zero(i, carry):
        cnt_v[pl.ds(i * L, L)] = jnp.zeros((L,), jnp.int32)
        return carry
    lax.fori_loop(0, 256 // L, zero, 0)

    def scan(g, carry):
        v = aid_v[pl.ds(g * L, L)]
        b = lax.shift_right_logical(v, 7)
        m = (b >= lo) & (b < hi)

        lbc = (b - lo) & 255
        dup, last = plsc.scan_count(lbc, m)
        cur = plsc.load_gather(cnt_v, [lbc], mask=m)
        slot = (cur + dup - 1) & (CAP - 1)
        addr = lbc * CAP + slot
        plsc.store_scatter(bk_ids, [addr], v, mask=m)
        plsc.store_scatter(bk_pos, [addr], g * L + iota, mask=m)
        plsc.addupdate_scatter(cnt_v, [lbc], dup, mask=m & last)
        return carry
    lax.fori_loop(0, B // L, scan, 0)

    cidx = [cc * L + iota for cc in range(D // L)]

    def blkcnt(j):
        return _lane(cnt_v[pl.ds((j >> 4) * L, L)], j & (L - 1))

    def row_drain(par, n):
        def w(i, carry):
            pltpu.make_async_copy(
                out_hbm.at[pl.ds(0, D)], rb.at[par, 0], osem0).wait()
            return carry

        def w1(i, carry):
            pltpu.make_async_copy(
                out_hbm.at[pl.ds(0, D)], rb.at[par, 0], osem1).wait()
            return carry

        def d0():
            lax.fori_loop(0, n, w, 0)

        def d1():
            lax.fori_loop(0, n, w1, 0)
        pl.when(par == 0)(d0)
        pl.when(par == 1)(d1)

    def per_block(j, carry):
        par = j & 1
        g = j >> 2
        at_ss_start = (j & (SS - 1)) == 0
        pl.when(at_ss_start & (g + RING - 1 < nss))(
            lambda: fire(g + RING - 1))
        pl.when(at_ss_start)(lambda: slab_wait(g))
        pl.when(j >= 2)(lambda: row_drain(par, blkcnt(j - 2)))

        n = blkcnt(j)

        def process():
            r = lax.rem(g, RING)
            lsub = (j - ssbase(g)) * VB
            v_id = bk_ids[pl.ds(j * CAP, CAP)]
            v_pos = bk_pos[pl.ds(j * CAP, CAP)]
            v_l = (v_id & (VB - 1)) + lsub

            def per_id(i, carry):
                spl = jnp.full((L,), _lane(v_l, i), jnp.int32)
                k = _lane(v_pos, i)
                for cc in range(D // L):
                    rb[par, i, pl.ds(cc * L, L)] = plsc.load_gather(
                        blk.at[r], [cidx[cc], spl])
                dst = out_hbm.at[pl.ds(pl.multiple_of(k * D, D), D)]

                def send0():
                    pltpu.async_copy(rb.at[0, i], dst, osem0)

                def send1():
                    pltpu.async_copy(rb.at[1, i], dst, osem1)
                pl.when(par == 0)(send0)
                pl.when(par == 1)(send1)
                return carry
            lax.fori_loop(0, n, per_id, 0)
        pl.when(n > 0)(process)
        return carry

    lax.fori_loop(0, nb, per_block, 0)

    row_drain(nb & 1, blkcnt(nb - 2))
    row_drain((nb - 1) & 1, blkcnt(nb - 1))


@jax.jit
def kernel(ids, embedding):
    ids_flat = jnp.reshape(ids, (B,)).astype(jnp.int32)
    table_t = jnp.transpose(embedding)
    run = pl.kernel(
        _body,
        out_type=jax.ShapeDtypeStruct((B * D,), jnp.float32),
        mesh=plsc.VectorSubcoreMesh(core_axis_name="c", subcore_axis_name="s"),
        scratch_types=[
            pltpu.VMEM((B,), jnp.int32),
            pltpu.VMEM((4096,), jnp.int32),
            pltpu.VMEM((4096,), jnp.int32),
            pltpu.VMEM((256,), jnp.int32),
            pltpu.VMEM((RING, D, SW), jnp.float32),
            pltpu.VMEM((2, CAP, D), jnp.float32),
            pltpu.SemaphoreType.DMA((RING,)),
            pltpu.SemaphoreType.DMA,
            pltpu.SemaphoreType.DMA,
        ],
        compiler_params=pltpu.CompilerParams(
            use_tc_tiling_on_sc=True, needs_layout_passes=False),
    )
    return jnp.reshape(run(ids_flat, table_t), (B, D))


# speedup vs baseline: 1.1596x; 1.1596x over previous
import jax
import jax.numpy as jnp
from jax import lax
from jax.experimental import pallas as pl
from jax.experimental.pallas import tpu as pltpu
from jax.experimental.pallas import tpu_sc as plsc

B = 16384
D = 64
V = 1_000_000
VB = 128
NB_TOT = (V + VB - 1) // VB
NC = 2
NS = 16
NW = NC * NS
NBW = (NB_TOT + NW - 1) // NW
CAP = 16
L = 16
NBUF = 8


def _lane(v, i):
    sel = jnp.where(lax.iota(jnp.int32, L) == i, v, 0)
    return lax.reduce_max(sel, (0,))


def _body(ids_hbm, table_hbm, out_hbm, aid_v, bk_ids, bk_pos, cnt_v,
          blk, rb, gsem, osem0, osem1):
    wid = lax.axis_index("s") * NC + lax.axis_index("c")
    lo = wid * NBW
    hi = jnp.minimum(lo + NBW, NB_TOT)
    nb = hi - lo
    iota = lax.iota(jnp.int32, L)

    def fire(j):
        off = pl.multiple_of((lo + j) * VB, VB)
        pltpu.async_copy(table_hbm.at[:, pl.ds(off, VB)],
                         blk.at[j & (NBUF - 1)], gsem.at[j & (NBUF - 1)])
        return None

    for jj in range(NBUF - 1):
        fire(jj)

    pltpu.sync_copy(ids_hbm, aid_v)

    def zero(i, carry):
        cnt_v[pl.ds(i * L, L)] = jnp.zeros((L,), jnp.int32)
        return carry
    lax.fori_loop(0, 256 // L, zero, 0)

    def scan(g, carry):
        v = aid_v[pl.ds(g * L, L)]
        b = lax.shift_right_logical(v, 7)
        m = (b >= lo) & (b < hi)

        lbc = (b - lo) & 255
        dup, last = plsc.scan_count(lbc, m)
        cur = plsc.load_gather(cnt_v, [lbc], mask=m)
        slot = (cur + dup - 1) & (CAP - 1)
        addr = lbc * CAP + slot
        plsc.store_scatter(bk_ids, [addr], v, mask=m)
        plsc.store_scatter(bk_pos, [addr], g * L + iota, mask=m)
        plsc.addupdate_scatter(cnt_v, [lbc], dup, mask=m & last)
        return carry
    lax.fori_loop(0, B // L, scan, 0)

    cidx = [cc * L + iota for cc in range(D // L)]

    def blkcnt(j):
        return _lane(cnt_v[pl.ds((j >> 4) * L, L)], j & (L - 1))

    def slab_wait(j):
        pltpu.make_async_copy(
            table_hbm.at[:, pl.ds(0, VB)], blk.at[j & (NBUF - 1)],
            gsem.at[j & (NBUF - 1)]).wait()
        return None

    def row_drain(par, n):
        def w(i, carry):
            pltpu.make_async_copy(
                out_hbm.at[pl.ds(0, D)], rb.at[par, 0], osem0).wait()
            return carry

        def w1(i, carry):
            pltpu.make_async_copy(
                out_hbm.at[pl.ds(0, D)], rb.at[par, 0], osem1).wait()
            return carry

        def d0():
            lax.fori_loop(0, n, w, 0)

        def d1():
            lax.fori_loop(0, n, w1, 0)
        pl.when(par == 0)(d0)
        pl.when(par == 1)(d1)

    def per_block(j, carry):
        par = j & 1
        pl.when((j + NBUF - 1 < nb) & (blkcnt(j + NBUF - 1) > 0))(
            lambda: fire(j + NBUF - 1))
        n = blkcnt(j)
        pl.when((j < NBUF - 1) | (n > 0))(lambda: slab_wait(j))
        pl.when(j >= 2)(lambda: row_drain(par, blkcnt(j - 2)))

        def process():
            bi = j & (NBUF - 1)
            v_id = bk_ids[pl.ds(j * CAP, CAP)]
            v_pos = bk_pos[pl.ds(j * CAP, CAP)]
            v_l = v_id & (VB - 1)

            def per_id(i, carry):
                spl = jnp.full((L,), _lane(v_l, i), jnp.int32)
                k = _lane(v_pos, i)
                for cc in range(D // L):
                    rb[par, i, pl.ds(cc * L, L)] = plsc.load_gather(
                        blk.at[bi], [cidx[cc], spl])
                dst = out_hbm.at[pl.ds(pl.multiple_of(k * D, D), D)]

                def send0():
                    pltpu.async_copy(rb.at[0, i], dst, osem0)

                def send1():
                    pltpu.async_copy(rb.at[1, i], dst, osem1)
                pl.when(par == 0)(send0)
                pl.when(par == 1)(send1)
                return carry
            lax.fori_loop(0, n, per_id, 0)
        pl.when(n > 0)(process)
        return carry

    lax.fori_loop(0, nb, per_block, 0)

    row_drain(nb & 1, blkcnt(nb - 2))
    row_drain((nb - 1) & 1, blkcnt(nb - 1))


@jax.jit
def kernel(ids, embedding):
    ids_flat = jnp.reshape(ids, (B,)).astype(jnp.int32)
    table_t = jnp.transpose(embedding)
    run = pl.kernel(
        _body,
        out_type=jax.ShapeDtypeStruct((B * D,), jnp.float32),
        mesh=plsc.VectorSubcoreMesh(core_axis_name="c", subcore_axis_name="s"),
        scratch_types=[
            pltpu.VMEM((B,), jnp.int32),
            pltpu.VMEM((4096,), jnp.int32),
            pltpu.VMEM((4096,), jnp.int32),
            pltpu.VMEM((256,), jnp.int32),
            pltpu.VMEM((NBUF, D, VB), jnp.float32),
            pltpu.VMEM((2, CAP, D), jnp.float32),
            pltpu.SemaphoreType.DMA((NBUF,)),
            pltpu.SemaphoreType.DMA,
            pltpu.SemaphoreType.DMA,
        ],
        compiler_params=pltpu.CompilerParams(
            use_tc_tiling_on_sc=True, needs_layout_passes=False),
    )
    return jnp.reshape(run(ids_flat, table_t), (B, D))


# speedup vs baseline: 1.1735x; 1.0120x over previous
import jax
import jax.numpy as jnp
from jax import lax
from jax.experimental import pallas as pl
from jax.experimental.pallas import tpu as pltpu
from jax.experimental.pallas import tpu_sc as plsc

B = 16384
D = 64
V = 1_000_000
VB = 128
NB_TOT = (V + VB - 1) // VB
NC = 2
NS = 16
NW = NC * NS
NBW = (NB_TOT + NW - 1) // NW
CAP = 16
L = 16
NBUF = 8


def _lane(v, i):
    sel = jnp.where(lax.iota(jnp.int32, L) == i, v, 0)
    return lax.reduce_max(sel, (0,))


def _body(ids_hbm, table_hbm, out_hbm, aid_v, bk_ids, bk_pos, cnt_v,
          blk, rb, gsem, osem):
    wid = lax.axis_index("s") * NC + lax.axis_index("c")
    lo = wid * NBW
    hi = jnp.minimum(lo + NBW, NB_TOT)
    nb = hi - lo
    iota = lax.iota(jnp.int32, L)

    def fire(j):
        off = pl.multiple_of((lo + j) * VB, VB)
        pltpu.async_copy(table_hbm.at[:, pl.ds(off, VB)],
                         blk.at[j & (NBUF - 1)], gsem.at[j & (NBUF - 1)])
        return None

    for jj in range(NBUF - 1):
        fire(jj)

    pltpu.sync_copy(ids_hbm, aid_v)

    def zero(i, carry):
        cnt_v[pl.ds(i * L, L)] = jnp.zeros((L,), jnp.int32)
        return carry
    lax.fori_loop(0, 256 // L, zero, 0)

    def scan(g, carry):
        v = aid_v[pl.ds(g * L, L)]
        b = lax.shift_right_logical(v, 7)
        m = (b >= lo) & (b < hi)

        lbc = (b - lo) & 255
        dup, last = plsc.scan_count(lbc, m)
        cur = plsc.load_gather(cnt_v, [lbc], mask=m)
        slot = (cur + dup - 1) & (CAP - 1)
        addr = lbc * CAP + slot
        plsc.store_scatter(bk_ids, [addr], v, mask=m)
        plsc.store_scatter(bk_pos, [addr], g * L + iota, mask=m)
        plsc.addupdate_scatter(cnt_v, [lbc], dup, mask=m & last)
        return carry
    lax.fori_loop(0, B // L, scan, 0)

    cidx = [cc * L + iota for cc in range(D // L)]

    def blkcnt(j):
        return _lane(cnt_v[pl.ds((j >> 4) * L, L)], j & (L - 1))

    def slab_wait(j):
        pltpu.make_async_copy(
            table_hbm.at[:, pl.ds(0, VB)], blk.at[j & (NBUF - 1)],
            gsem.at[j & (NBUF - 1)]).wait()
        return None

    def row_drain(par, n):
        def w(i, carry):
            pltpu.make_async_copy(
                out_hbm.at[pl.ds(0, D)], rb.at[par, 0], osem.at[par]).wait()
            return carry
        lax.fori_loop(0, n, w, 0)

    def per_block(j, carry):
        par = j & 1
        pl.when((j + NBUF - 1 < nb) & (blkcnt(j + NBUF - 1) > 0))(
            lambda: fire(j + NBUF - 1))
        n = blkcnt(j)
        pl.when((j < NBUF - 1) | (n > 0))(lambda: slab_wait(j))
        pl.when(j >= 2)(lambda: row_drain(par, blkcnt(j - 2)))

        def process():
            bi = j & (NBUF - 1)
            v_id = bk_ids[pl.ds(j * CAP, CAP)]
            v_pos = bk_pos[pl.ds(j * CAP, CAP)]
            v_l = v_id & (VB - 1)

            def per_id(i, carry):
                spl = jnp.full((L,), _lane(v_l, i), jnp.int32)
                k = _lane(v_pos, i)
                for cc in range(D // L):
                    rb[par, i, pl.ds(cc * L, L)] = plsc.load_gather(
                        blk.at[bi], [cidx[cc], spl])
                dst = out_hbm.at[pl.ds(pl.multiple_of(k * D, D), D)]
                pltpu.async_copy(rb.at[par, i], dst, osem.at[par])
                return carry
            lax.fori_loop(0, n, per_id, 0)
        pl.when(n > 0)(process)
        return carry

    lax.fori_loop(0, nb, per_block, 0)

    row_drain(nb & 1, blkcnt(nb - 2))
    row_drain((nb - 1) & 1, blkcnt(nb - 1))


@jax.jit
def kernel(ids, embedding):
    ids_flat = jnp.reshape(ids, (B,)).astype(jnp.int32)
    table_t = jnp.transpose(embedding)
    run = pl.kernel(
        _body,
        out_type=jax.ShapeDtypeStruct((B * D,), jnp.float32),
        mesh=plsc.VectorSubcoreMesh(core_axis_name="c", subcore_axis_name="s"),
        scratch_types=[
            pltpu.VMEM((B,), jnp.int32),
            pltpu.VMEM((4096,), jnp.int32),
            pltpu.VMEM((4096,), jnp.int32),
            pltpu.VMEM((256,), jnp.int32),
            pltpu.VMEM((NBUF, D, VB), jnp.float32),
            pltpu.VMEM((2, CAP, D), jnp.float32),
            pltpu.SemaphoreType.DMA((NBUF,)),
            pltpu.SemaphoreType.DMA((2,)),
        ],
        compiler_params=pltpu.CompilerParams(
            use_tc_tiling_on_sc=True, needs_layout_passes=False),
    )
    return jnp.reshape(run(ids_flat, table_t), (B, D))
